# baseline (speedup 1.0000x reference)
import numpy as np
from itertools import combinations

import jax
import jax.numpy as jnp
from jax import lax
from jax.experimental import pallas as pl
from jax.experimental.pallas import tpu as pltpu
from jax.experimental.pallas import tpu_sc as plsc

_SIZE_IN = 14
_HOTNESS = 3
_BATCH = 4096
_IN_DIM = 2 ** _SIZE_IN
_NSEL = 364

_FILT = np.array([sum(2 ** i for i in c) for c in combinations(range(_SIZE_IN), _HOTNESS)],
                 dtype=np.int32)
_GRAN = np.unique(_FILT >> 4)
_NG = len(_GRAN)
_GSLOT = {int(g): i for i, g in enumerate(_GRAN)}

_ROWS = np.array([_GSLOT[int(f) >> 4] for f in _FILT], dtype=np.int32)
_COLS = (_FILT & 15).astype(np.int32)

_NVEC = 23
_OFFS = [16 * v for v in range(_NVEC - 1)] + [_NSEL - 16]
_CROW = np.stack([_ROWS[o:o + 16] for o in _OFFS]).astype(np.int32)
_CCOL = np.stack([_COLS[o:o + 16] for o in _OFFS]).astype(np.int32)

_NCORES = 2
_NSUB = 16
_NTILES = _NCORES * _NSUB
_ROWS_PER_TILE = _BATCH // _NTILES
_BC = 16
_NCHUNK = _ROWS_PER_TILE // _BC
_CH_IDX = _BC * _NG
_CH_OUT = _BC * _NSEL


def _body(table_hbm, idx_hbm, crow_hbm, ccol_hbm, out_hbm,
          idxbuf, gbuf, crow_v, ccol_v, obuf, sems):
    wid = lax.axis_index("s") * _NCORES + lax.axis_index("c")
    pltpu.sync_copy(crow_hbm, crow_v)
    pltpu.sync_copy(ccol_hbm, ccol_v)

    def start_gather(c, buf):
        b0 = wid * _ROWS_PER_TILE + c * _BC
        pltpu.sync_copy(idx_hbm.at[pl.ds(b0 * _NG, _CH_IDX)], idxbuf.at[buf])
        return pltpu.async_copy(table_hbm.at[idxbuf.at[buf]], gbuf.at[buf],
                                sems.at[buf])

    start_gather(0, 0).wait()

    def chunk_body(c, _):
        buf = lax.rem(c, 2)
        nbuf = 1 - buf

        @pl.when(c + 1 < _NCHUNK)
        def _():
            start_gather(c + 1, nbuf)

        def row_body(r, _):
            rowbase = r * _NG
            for v in range(_NVEC):
                ir = crow_v[v] + rowbase
                ic = ccol_v[v]
                x = plsc.load_gather(gbuf.at[buf], [ir, ic])
                y = x * 12.0 - 6.0
                obuf[pl.ds(r * _NSEL + _OFFS[v], 16)] = y
            return 0

        lax.fori_loop(0, _BC, row_body, 0, unroll=False)
        b0 = wid * _ROWS_PER_TILE + c * _BC
        pltpu.sync_copy(obuf, out_hbm.at[pl.ds(b0 * _NSEL, _CH_OUT)])

        @pl.when(c + 1 < _NCHUNK)
        def _():
            pltpu.make_async_copy(table_hbm.at[idxbuf.at[nbuf]],
                                  gbuf.at[nbuf], sems.at[nbuf]).wait()
        return 0

    lax.fori_loop(0, _NCHUNK, chunk_body, 0, unroll=False)


def kernel(input_var):
    table = input_var.reshape(_BATCH * (_IN_DIM // 16), 16)
    idx = (jnp.arange(_BATCH, dtype=jnp.int32)[:, None] * (_IN_DIM // 16)
           + jnp.asarray(_GRAN, dtype=jnp.int32)[None, :]).reshape(-1)
    crow = jnp.asarray(_CROW)
    ccol = jnp.asarray(_CCOL)

    mesh = plsc.VectorSubcoreMesh(core_axis_name="c", subcore_axis_name="s",
                                  num_cores=_NCORES, num_subcores=_NSUB)
    out_flat = pl.kernel(
        _body,
        out_type=jax.ShapeDtypeStruct((_BATCH * _NSEL,), jnp.float32),
        mesh=mesh,
        scratch_types=[
            pltpu.VMEM((2, _CH_IDX), jnp.int32),
            pltpu.VMEM((2, _CH_IDX, 16), jnp.float32),
            pltpu.VMEM((_NVEC, 16), jnp.int32),
            pltpu.VMEM((_NVEC, 16), jnp.int32),
            pltpu.VMEM((_CH_OUT,), jnp.float32),
            pltpu.SemaphoreType.DMA((2,)),
        ],
        compiler_params=pltpu.CompilerParams(needs_layout_passes=False,
                                             use_tc_tiling_on_sc=False),
    )(table, idx, crow, ccol)
    return out_flat.reshape(_BATCH, _NSEL)
